# baseline (speedup 1.0000x reference)
import functools
import math

import jax
import jax.numpy as jnp
from jax.experimental import pallas as pl
from jax.experimental.pallas import tpu as pltpu

K_NEIGHBORS = 8


def _dgn_kernel(params_ref, q_ref, qh_ref, ql_ref, kh_ref, kl_ref,
                gain_ref, bias_ref, o_ref, *, bq, w, q0):
    i = pl.program_id(1)
    q = q_ref[0]
    qh = qh_ref[0]
    ql = ql_ref[0]
    kh = kh_ref[0]
    kl = kl_ref[0]

    dn = (((1,), (1,)), ((), ()))
    scores = jax.lax.dot_general(qh, kh, dn, preferred_element_type=jnp.float32)
    scores += jax.lax.dot_general(qh, kl, dn, preferred_element_type=jnp.float32)
    scores += jax.lax.dot_general(ql, kh, dn, preferred_element_type=jnp.float32)

    neg = jnp.finfo(jnp.float32).min
    rows = q0 + i * bq + jax.lax.broadcasted_iota(jnp.int32, (bq, w), 0)
    cols = jax.lax.broadcasted_iota(jnp.int32, (bq, w), 1)
    scores = jnp.where(cols <= rows, scores, neg)

    v = jnp.max(scores, axis=1, keepdims=True)
    for _ in range(K_NEIGHBORS - 1):
        v = jnp.max(jnp.where(scores < v, scores, neg), axis=1, keepdims=True)

    lim = jnp.maximum(v, jnp.float32(-1e37))
    adj = (scores >= lim).astype(jnp.bfloat16)

    msg = jax.lax.dot_general(
        adj, kh, (((1,), (0,)), ((), ())),
        preferred_element_type=jnp.float32)

    row_ids = q0 + i * bq + jax.lax.broadcasted_iota(jnp.int32, (bq, 1), 0)
    deg = jnp.minimum(row_ids + 1, K_NEIGHBORS).astype(jnp.float32)
    msg = msg / deg

    mix = params_ref[0]
    scale = params_ref[1]
    blended = mix * q + (1.0 - mix) * msg
    z = blended * gain_ref[...] + bias_ref[...]
    delta = 0.5 * z * (1.0 + jax.lax.erf(z / math.sqrt(2.0))) * scale
    o_ref[0] = delta


@jax.jit
def kernel(x, gain, bias, log_mix, log_scale):
    b, t, d = x.shape
    bq = 256
    mix = jax.nn.sigmoid(log_mix)
    scale = jax.nn.softplus(log_scale) + 0.01
    params = jnp.stack([mix, scale]).astype(jnp.float32)
    xh = x.astype(jnp.bfloat16)
    xl = (x - xh.astype(jnp.float32)).astype(jnp.bfloat16)

    n_groups = 4
    panels_per_group = t // bq // n_groups
    gq = panels_per_group * bq
    outs = []
    for g in range(n_groups):
        q0 = g * gq
        w = (g + 1) * gq
        grid = (b, panels_per_group)

        def q_map(bi, qi, _g=g, _p=panels_per_group):
            return (bi, _g * _p + qi, 0)

        out = pl.pallas_call(
            functools.partial(_dgn_kernel, bq=bq, w=w, q0=q0),
            grid=grid,
            in_specs=[
                pl.BlockSpec(memory_space=pltpu.SMEM),
                pl.BlockSpec((1, bq, d), q_map),
                pl.BlockSpec((1, bq, d), q_map),
                pl.BlockSpec((1, bq, d), q_map),
                pl.BlockSpec((1, w, d), lambda bi, qi: (bi, 0, 0)),
                pl.BlockSpec((1, w, d), lambda bi, qi: (bi, 0, 0)),
                pl.BlockSpec((d,), lambda bi, qi: (0,)),
                pl.BlockSpec((d,), lambda bi, qi: (0,)),
            ],
            out_specs=pl.BlockSpec((1, bq, d), lambda bi, qi: (bi, qi, 0)),
            out_shape=jax.ShapeDtypeStruct((b, gq, d), jnp.float32),
        )(params, x, xh, xl, xh, xl, gain, bias)
        outs.append(out)
    return jnp.concatenate(outs, axis=1)


# speedup vs baseline: 1.2714x; 1.2714x over previous
import functools
import math

import jax
import jax.numpy as jnp
from jax.experimental import pallas as pl
from jax.experimental.pallas import tpu as pltpu

K_NEIGHBORS = 8


def _dgn_kernel(params_ref, q_ref, k_ref, kb_ref, gain_ref, bias_ref, o_ref,
                *, bq, w, q0):
    i = pl.program_id(1)
    q = q_ref[0]
    keys = k_ref[0]

    scores = jax.lax.dot_general(
        q, keys, (((1,), (1,)), ((), ())),
        preferred_element_type=jnp.float32)

    neg = jnp.finfo(jnp.float32).min
    rows = q0 + i * bq + jax.lax.broadcasted_iota(jnp.int32, (bq, w), 0)
    cols = jax.lax.broadcasted_iota(jnp.int32, (bq, w), 1)
    scores = jnp.where(cols <= rows, scores, neg)

    v = jnp.max(scores, axis=1, keepdims=True)
    for _ in range(K_NEIGHBORS - 1):
        v = jnp.max(jnp.where(scores < v, scores, neg), axis=1, keepdims=True)

    lim = jnp.maximum(v, jnp.float32(-1e37))
    adj = (scores >= lim).astype(jnp.bfloat16)

    msg = jax.lax.dot_general(
        adj, kb_ref[0], (((1,), (0,)), ((), ())),
        preferred_element_type=jnp.float32)

    row_ids = q0 + i * bq + jax.lax.broadcasted_iota(jnp.int32, (bq, 1), 0)
    deg = jnp.minimum(row_ids + 1, K_NEIGHBORS).astype(jnp.float32)
    msg = msg / deg

    mix = params_ref[0]
    scale = params_ref[1]
    blended = mix * q + (1.0 - mix) * msg
    z = blended * gain_ref[...] + bias_ref[...]
    delta = 0.5 * z * (1.0 + jax.lax.erf(z / math.sqrt(2.0))) * scale
    o_ref[0] = delta


@jax.jit
def kernel(x, gain, bias, log_mix, log_scale):
    b, t, d = x.shape
    bq = 256
    mix = jax.nn.sigmoid(log_mix)
    scale = jax.nn.softplus(log_scale) + 0.01
    params = jnp.stack([mix, scale]).astype(jnp.float32)
    xb = x.astype(jnp.bfloat16)

    n_groups = 4
    panels_per_group = t // bq // n_groups
    gq = panels_per_group * bq
    outs = []
    for g in range(n_groups):
        q0 = g * gq
        w = (g + 1) * gq
        grid = (b, panels_per_group)

        def q_map(bi, qi, _g=g, _p=panels_per_group):
            return (bi, _g * _p + qi, 0)

        out = pl.pallas_call(
            functools.partial(_dgn_kernel, bq=bq, w=w, q0=q0),
            grid=grid,
            in_specs=[
                pl.BlockSpec(memory_space=pltpu.SMEM),
                pl.BlockSpec((1, bq, d), q_map),
                pl.BlockSpec((1, w, d), lambda bi, qi: (bi, 0, 0)),
                pl.BlockSpec((1, w, d), lambda bi, qi: (bi, 0, 0)),
                pl.BlockSpec((d,), lambda bi, qi: (0,)),
                pl.BlockSpec((d,), lambda bi, qi: (0,)),
            ],
            out_specs=pl.BlockSpec((1, bq, d), lambda bi, qi: (bi, qi, 0)),
            out_shape=jax.ShapeDtypeStruct((b, gq, d), jnp.float32),
        )(params, x, x, xb, gain, bias)
        outs.append(out)
    return jnp.concatenate(outs, axis=1)
